# baseline (speedup 1.0000x reference)
import functools

import jax
import jax.numpy as jnp
from jax import lax
from jax.experimental import pallas as pl
from jax.experimental.pallas import tpu as pltpu
from jax.experimental.pallas import tpu_sc as plsc

N = 10000
E = 320000
D = 128

NC = 2
NS = 16
NW = NC * NS
EDGES_PER_TILE = E // NW
CHUNK = 80
NCHUNK = 128
EP = NCHUNK * CHUNK
GPH = NCHUNK // 2
SPH = NCHUNK // 4
NPAD = N + 8
STRIPE = 632
LAST_STRIPE = NPAD - (NS - 1) * STRIPE


@functools.partial(
    pl.kernel,
    out_type=jax.ShapeDtypeStruct((NC, NPAD, D), jnp.float32),
    mesh=plsc.VectorSubcoreMesh(core_axis_name="c", subcore_axis_name="s"),
    scratch_types=[
        pltpu.VMEM((GPH * CHUNK,), jnp.int32),
        pltpu.VMEM((SPH, CHUNK), jnp.int32),
        pltpu.VMEM((CHUNK, D), jnp.float32),
        pltpu.VMEM((CHUNK, D), jnp.float32),
        pltpu.VMEM((CHUNK, D), jnp.float32),
        pltpu.VMEM((CHUNK, D), jnp.float32),
        pltpu.SemaphoreType.DMA,
        pltpu.SemaphoreType.DMA,
        pltpu.SemaphoreType.DMA,
        pltpu.SemaphoreType.DMA,
        pltpu.SemaphoreType.DMA,
        pltpu.SemaphoreType.DMA,
        pltpu.SemaphoreType.DMA,
        pltpu.SemaphoreType.DMA,
        pltpu.VMEM_SHARED((NPAD, D), jnp.float32),
    ],
)
def _seg_sum(h_hbm, src_hbm, dst_hbm, zeros_hbm, out_hbm,
             sidx, didx, rows0, rows1, rows2, rows3,
             gsem0, gsem1, gsem2, gsem3, ssem0, ssem1, ssem2, ssem3, acc):
    c = lax.axis_index("c")
    s = lax.axis_index("s")
    wid = c * NS + s
    pltpu.sync_copy(src_hbm.at[wid, 0], sidx)
    pltpu.sync_copy(dst_hbm.at[wid, 0], didx)

    rows = (rows0, rows1, rows2, rows3)
    gsems = (gsem0, gsem1, gsem2, gsem3)
    ssems = (ssem0, ssem1, ssem2, ssem3)

    def gather(jg, b):
        pltpu.async_copy(h_hbm.at[sidx.at[pl.ds(jg * CHUNK, CHUNK)]],
                         rows[b], gsems[b])

    def wait_gather(b):
        pltpu.make_async_copy(h_hbm.at[sidx.at[pl.ds(0, CHUNK)]],
                              rows[b], gsems[b]).wait()

    def scatter(js, b):
        pltpu.async_copy(rows[b], acc.at[didx.at[js]], ssems[b], add=True)

    def wait_scatter(b):
        pltpu.make_async_copy(rows[b], acc.at[didx.at[0]], ssems[b]).wait()

    gather(0, 0)
    gather(1, 1)
    gather(2, 2)

    @pl.when(s < NS - 1)
    def _():
        pltpu.sync_copy(zeros_hbm, acc.at[pl.ds(s * STRIPE, STRIPE)])

    @pl.when(s == NS - 1)
    def _():
        pltpu.sync_copy(zeros_hbm.at[pl.ds(0, LAST_STRIPE)],
                        acc.at[pl.ds((NS - 1) * STRIPE, LAST_STRIPE)])

    plsc.subcore_barrier()

    def step(b, js, wait_prev, jg):
        wait_gather(b)
        scatter(js, b)
        if wait_prev:
            wait_scatter((b + 3) % 4)
        if jg is not None:
            gather(jg, (b + 3) % 4)

    def loop28(first_i, soff, goff):
        b0 = first_i % 4

        def body(p, carry):
            i = first_i + 4 * p
            for k in range(4):
                step((b0 + k) % 4, i + k - soff, True, i + k + 3 - goff)
            return carry

        lax.fori_loop(0, 7, body, 0)

    step(0, 0, False, 3)
    step(1, 1, True, 4)
    step(2, 2, True, 5)
    loop28(3, 0, 0)
    step(3, 31, True, 34)
    wait_scatter(3)
    pltpu.sync_copy(dst_hbm.at[wid, 1], didx)
    step(0, 0, False, 35)
    loop28(33, 32, 0)
    step(1, 29, True, None)
    step(2, 30, True, None)
    step(3, 31, True, None)
    wait_scatter(3)
    pltpu.sync_copy(src_hbm.at[wid, 1], sidx)
    pltpu.sync_copy(dst_hbm.at[wid, 2], didx)
    gather(0, 0)
    gather(1, 1)
    gather(2, 2)
    step(0, 0, False, 3)
    step(1, 1, True, 4)
    step(2, 2, True, 5)
    loop28(67, 64, 64)
    step(3, 31, True, 34)
    wait_scatter(3)
    pltpu.sync_copy(dst_hbm.at[wid, 3], didx)
    step(0, 0, False, 35)
    loop28(97, 96, 64)
    step(1, 29, True, None)
    step(2, 30, True, None)
    step(3, 31, True, None)
    wait_scatter(3)
    plsc.subcore_barrier()

    @pl.when(s < NS - 1)
    def _():
        pltpu.sync_copy(acc.at[pl.ds(s * STRIPE, STRIPE)],
                        out_hbm.at[c, pl.ds(s * STRIPE, STRIPE)])

    @pl.when(s == NS - 1)
    def _():
        pltpu.sync_copy(acc.at[pl.ds((NS - 1) * STRIPE, LAST_STRIPE)],
                        out_hbm.at[c, pl.ds((NS - 1) * STRIPE, LAST_STRIPE)])


BR = 2000


def _mlp_body(part_ref, h_ref, w1_ref, b1_ref, w2_ref, b2_ref, out_ref):
    z = h_ref[...] + part_ref[0] + part_ref[1]
    z1 = jnp.maximum(
        jnp.dot(z, w1_ref[...], preferred_element_type=jnp.float32) + b1_ref[...],
        0.0)
    z2 = jnp.dot(z1, w2_ref[...], preferred_element_type=jnp.float32) + b2_ref[...]
    out_ref[...] = jnp.maximum(z2, 0.0)


_mlp = pl.pallas_call(
    _mlp_body,
    grid=(N // BR,),
    in_specs=[
        pl.BlockSpec((NC, BR, D), lambda i: (0, i, 0)),
        pl.BlockSpec((BR, D), lambda i: (i, 0)),
        pl.BlockSpec((D, D), lambda i: (0, 0)),
        pl.BlockSpec((1, D), lambda i: (0, 0)),
        pl.BlockSpec((D, D), lambda i: (0, 0)),
        pl.BlockSpec((1, D), lambda i: (0, 0)),
    ],
    out_specs=pl.BlockSpec((BR, D), lambda i: (i, 0)),
    out_shape=jax.ShapeDtypeStruct((N, D), jnp.float32),
)


def _mlp_final_body(part_ref, h_ref, w1_ref, b1_ref, w2_ref, b2_ref,
                    wl_ref, bl_ref, out_ref):
    z = h_ref[...] + part_ref[0] + part_ref[1]
    z1 = jnp.maximum(
        jnp.dot(z, w1_ref[...], preferred_element_type=jnp.float32) + b1_ref[...],
        0.0)
    z2 = jnp.dot(z1, w2_ref[...], preferred_element_type=jnp.float32) + b2_ref[...]
    h3 = jnp.maximum(z2, 0.0)
    logit = jnp.dot(h3, wl_ref[...], preferred_element_type=jnp.float32) + bl_ref[...]
    out_ref[...] = 1.0 / (1.0 + jnp.exp(-logit))


_mlp_final = pl.pallas_call(
    _mlp_final_body,
    grid=(N // BR,),
    in_specs=[
        pl.BlockSpec((NC, BR, D), lambda i: (0, i, 0)),
        pl.BlockSpec((BR, D), lambda i: (i, 0)),
        pl.BlockSpec((D, D), lambda i: (0, 0)),
        pl.BlockSpec((1, D), lambda i: (0, 0)),
        pl.BlockSpec((D, D), lambda i: (0, 0)),
        pl.BlockSpec((1, D), lambda i: (0, 0)),
        pl.BlockSpec((D, 1), lambda i: (0, 0)),
        pl.BlockSpec((1, 1), lambda i: (0, 0)),
    ],
    out_specs=pl.BlockSpec((BR, 1), lambda i: (i, 0)),
    out_shape=jax.ShapeDtypeStruct((N, 1), jnp.float32),
)


def kernel(x, edge_index, W1_0, b1_0, W2_0, b2_0, W1_1, b1_1, W2_1, b2_1,
           W1_2, b1_2, W2_2, b2_2, Wl, bl):
    src = edge_index[0].reshape(NW, EDGES_PER_TILE)
    dst = edge_index[1].reshape(NW, EDGES_PER_TILE)
    pad = EP - EDGES_PER_TILE
    srcf = jnp.concatenate(
        [src, jnp.zeros((NW, pad), jnp.int32)],
        axis=1).reshape(NW, 2, GPH * CHUNK)
    dst3 = jnp.concatenate(
        [dst, jnp.full((NW, pad), N, jnp.int32)],
        axis=1).reshape(NW, 4, SPH, CHUNK)
    zeros = jnp.zeros((STRIPE, D), jnp.float32)
    params = [(W1_0, b1_0, W2_0, b2_0), (W1_1, b1_1, W2_1, b2_1),
              (W1_2, b1_2, W2_2, b2_2)]
    h = x
    for li, (W1, b1, W2, b2) in enumerate(params):
        part = _seg_sum(h, srcf, dst3, zeros)
        b1r = b1.reshape(1, D)
        b2r = b2.reshape(1, D)
        if li < 2:
            h = _mlp(part, h, W1, b1r, W2, b2r)
        else:
            out = _mlp_final(part, h, W1, b1r, W2, b2r, Wl, bl.reshape(1, 1))
    return out[:, 0]


# speedup vs baseline: 1.9145x; 1.9145x over previous
import functools

import jax
import jax.numpy as jnp
from jax import lax
from jax.experimental import pallas as pl
from jax.experimental.pallas import tpu as pltpu
from jax.experimental.pallas import tpu_sc as plsc

N = 10000
E = 320000
D = 128

NC = 2
NS = 16
NW = NC * NS
EDGES_PER_TILE = E // NW
CHUNK = 80
NCHUNK = 126
EP = NCHUNK * CHUNK
NPAD = N + 8
STRIPE = 632
LAST_STRIPE = NPAD - (NS - 1) * STRIPE


@functools.partial(
    pl.kernel,
    out_type=jax.ShapeDtypeStruct((NC, NPAD, D), jnp.float32),
    mesh=plsc.VectorSubcoreMesh(core_axis_name="c", subcore_axis_name="s"),
    scratch_types=[
        pltpu.VMEM((EP,), jnp.int32),
        pltpu.VMEM((NCHUNK // 2, CHUNK), jnp.int32),
        pltpu.VMEM((CHUNK, D), jnp.float32),
        pltpu.VMEM((CHUNK, D), jnp.float32),
        pltpu.VMEM((CHUNK, D), jnp.float32),
        pltpu.SemaphoreType.DMA,
        pltpu.SemaphoreType.DMA,
        pltpu.SemaphoreType.DMA,
        pltpu.SemaphoreType.DMA,
        pltpu.SemaphoreType.DMA,
        pltpu.SemaphoreType.DMA,
        pltpu.VMEM_SHARED((NPAD, D), jnp.float32),
    ],
)
def _seg_sum(h_hbm, src_hbm, dst_hbm, zeros_hbm, out_hbm,
             sidx, didx, rows0, rows1, rows2,
             gsem0, gsem1, gsem2, ssem0, ssem1, ssem2, acc):
    c = lax.axis_index("c")
    s = lax.axis_index("s")
    wid = c * NS + s
    pltpu.sync_copy(src_hbm.at[wid], sidx)
    pltpu.sync_copy(dst_hbm.at[wid, 0], didx)

    rows = (rows0, rows1, rows2)
    gsems = (gsem0, gsem1, gsem2)
    ssems = (ssem0, ssem1, ssem2)

    def gather(i, b):
        pltpu.async_copy(h_hbm.at[sidx.at[pl.ds(i * CHUNK, CHUNK)]],
                         rows[b], gsems[b])

    def wait_gather(b):
        pltpu.make_async_copy(h_hbm.at[sidx.at[pl.ds(0, CHUNK)]],
                              rows[b], gsems[b]).wait()

    def scatter(j, b):
        pltpu.async_copy(rows[b], acc.at[didx.at[j]], ssems[b], add=True)

    def wait_scatter(b):
        pltpu.make_async_copy(rows[b], acc.at[didx.at[0]], ssems[b]).wait()

    gather(0, 0)
    gather(1, 1)

    @pl.when(s < NS - 1)
    def _():
        pltpu.sync_copy(zeros_hbm, acc.at[pl.ds(s * STRIPE, STRIPE)])

    @pl.when(s == NS - 1)
    def _():
        pltpu.sync_copy(zeros_hbm.at[pl.ds(0, LAST_STRIPE)],
                        acc.at[pl.ds((NS - 1) * STRIPE, LAST_STRIPE)])

    plsc.subcore_barrier()

    def step(i, j, b, wait_prev, prefetch):
        wait_gather(b)
        scatter(j, b)
        if wait_prev:
            wait_scatter((b + 2) % 3)
        if prefetch:
            gather(i + 2, (b + 2) % 3)

    step(0, 0, 0, False, True)
    step(1, 1, 1, True, True)

    def body1(p, carry):
        i = 2 + 3 * p
        step(i, i, 2, True, True)
        step(i + 1, i + 1, 0, True, True)
        step(i + 2, i + 2, 1, True, True)
        return carry

    lax.fori_loop(0, 20, body1, 0)
    step(62, 62, 2, True, True)
    wait_scatter(2)
    pltpu.sync_copy(dst_hbm.at[wid, 1], didx)

    step(63, 0, 0, False, True)
    step(64, 1, 1, True, True)

    def body2(p, carry):
        i = 65 + 3 * p
        step(i, i - 63, 2, True, True)
        step(i + 1, i - 62, 0, True, True)
        step(i + 2, i - 61, 1, True, True)
        return carry

    lax.fori_loop(0, 19, body2, 0)
    step(122, 59, 2, True, True)
    step(123, 60, 0, True, True)
    step(124, 61, 1, True, False)
    step(125, 62, 2, True, False)
    wait_scatter(2)
    plsc.subcore_barrier()

    @pl.when(s < NS - 1)
    def _():
        pltpu.sync_copy(acc.at[pl.ds(s * STRIPE, STRIPE)],
                        out_hbm.at[c, pl.ds(s * STRIPE, STRIPE)])

    @pl.when(s == NS - 1)
    def _():
        pltpu.sync_copy(acc.at[pl.ds((NS - 1) * STRIPE, LAST_STRIPE)],
                        out_hbm.at[c, pl.ds((NS - 1) * STRIPE, LAST_STRIPE)])


BR = 2000


def _mlp_body(part_ref, h_ref, w1_ref, b1_ref, w2_ref, b2_ref, out_ref):
    z = h_ref[...] + part_ref[0] + part_ref[1]
    z1 = jnp.maximum(
        jnp.dot(z, w1_ref[...], preferred_element_type=jnp.float32) + b1_ref[...],
        0.0)
    z2 = jnp.dot(z1, w2_ref[...], preferred_element_type=jnp.float32) + b2_ref[...]
    out_ref[...] = jnp.maximum(z2, 0.0)


_mlp = pl.pallas_call(
    _mlp_body,
    grid=(N // BR,),
    in_specs=[
        pl.BlockSpec((NC, BR, D), lambda i: (0, i, 0)),
        pl.BlockSpec((BR, D), lambda i: (i, 0)),
        pl.BlockSpec((D, D), lambda i: (0, 0)),
        pl.BlockSpec((1, D), lambda i: (0, 0)),
        pl.BlockSpec((D, D), lambda i: (0, 0)),
        pl.BlockSpec((1, D), lambda i: (0, 0)),
    ],
    out_specs=pl.BlockSpec((BR, D), lambda i: (i, 0)),
    out_shape=jax.ShapeDtypeStruct((N, D), jnp.float32),
)


def _mlp_final_body(part_ref, h_ref, w1_ref, b1_ref, w2_ref, b2_ref,
                    wl_ref, bl_ref, out_ref):
    z = h_ref[...] + part_ref[0] + part_ref[1]
    z1 = jnp.maximum(
        jnp.dot(z, w1_ref[...], preferred_element_type=jnp.float32) + b1_ref[...],
        0.0)
    z2 = jnp.dot(z1, w2_ref[...], preferred_element_type=jnp.float32) + b2_ref[...]
    h3 = jnp.maximum(z2, 0.0)
    logit = jnp.dot(h3, wl_ref[...], preferred_element_type=jnp.float32) + bl_ref[...]
    out_ref[...] = 1.0 / (1.0 + jnp.exp(-logit))


_mlp_final = pl.pallas_call(
    _mlp_final_body,
    grid=(N // BR,),
    in_specs=[
        pl.BlockSpec((NC, BR, D), lambda i: (0, i, 0)),
        pl.BlockSpec((BR, D), lambda i: (i, 0)),
        pl.BlockSpec((D, D), lambda i: (0, 0)),
        pl.BlockSpec((1, D), lambda i: (0, 0)),
        pl.BlockSpec((D, D), lambda i: (0, 0)),
        pl.BlockSpec((1, D), lambda i: (0, 0)),
        pl.BlockSpec((D, 1), lambda i: (0, 0)),
        pl.BlockSpec((1, 1), lambda i: (0, 0)),
    ],
    out_specs=pl.BlockSpec((BR, 1), lambda i: (i, 0)),
    out_shape=jax.ShapeDtypeStruct((N, 1), jnp.float32),
)


def kernel(x, edge_index, W1_0, b1_0, W2_0, b2_0, W1_1, b1_1, W2_1, b2_1,
           W1_2, b1_2, W2_2, b2_2, Wl, bl):
    src = edge_index[0].reshape(NW, EDGES_PER_TILE)
    dst = edge_index[1].reshape(NW, EDGES_PER_TILE)
    pad = EP - EDGES_PER_TILE
    srcf = jnp.concatenate(
        [src, jnp.zeros((NW, pad), jnp.int32)], axis=1)
    dst3 = jnp.concatenate(
        [dst, jnp.full((NW, pad), N, jnp.int32)],
        axis=1).reshape(NW, 2, NCHUNK // 2, CHUNK)
    zeros = jnp.zeros((STRIPE, D), jnp.float32)
    params = [(W1_0, b1_0, W2_0, b2_0), (W1_1, b1_1, W2_1, b2_1),
              (W1_2, b1_2, W2_2, b2_2)]
    h = x
    for li, (W1, b1, W2, b2) in enumerate(params):
        part = _seg_sum(h, srcf, dst3, zeros)
        b1r = b1.reshape(1, D)
        b2r = b2.reshape(1, D)
        if li < 2:
            h = _mlp(part, h, W1, b1r, W2, b2r)
        else:
            out = _mlp_final(part, h, W1, b1r, W2, b2r, Wl, bl.reshape(1, 1))
    return out[:, 0]
